# baseline (speedup 1.0000x reference)
import functools

import jax
import jax.numpy as jnp
from jax import lax
from jax.experimental import pallas as pl
from jax.experimental.pallas import tpu as pltpu
from jax.experimental.pallas import tpu_sc as plsc

NMP = 3
NROW = 2048
D = 64
K = 32
ROWB = 256
NJB = NROW // ROWB
NW = 32
RPW = K * NROW // NW
CHUNK = 128
NCH = RPW // CHUNK
IDXROWS = NMP * K * NROW // CHUNK
HW = D // 2


def _pack_words(pd, dst):
    lo = jax.lax.bitcast_convert_type(pd.astype(jnp.bfloat16), jnp.uint16).astype(jnp.uint32)
    hi = jax.lax.bitcast_convert_type(dst.astype(jnp.bfloat16), jnp.uint16).astype(jnp.uint32)
    return jax.lax.bitcast_convert_type(lo | (hi << 16), jnp.int32)


def _unpack_lo(wds):
    u = jax.lax.bitcast_convert_type(wds, jnp.uint32)
    return jax.lax.bitcast_convert_type((u & 0xFFFF).astype(jnp.uint16),
                                        jnp.bfloat16).astype(jnp.float32)


def _unpack_hi(wds):
    u = jax.lax.bitcast_convert_type(wds, jnp.uint32)
    return jax.lax.bitcast_convert_type((u >> 16).astype(jnp.uint16),
                                        jnp.bfloat16).astype(jnp.float32)


def _proj_body(dst_ref, Wp_ref, Bp_ref, c_ref):
    dst = dst_ref[...]
    pd = jnp.dot(dst, Wp_ref[0], preferred_element_type=jnp.float32) + Bp_ref[0, 0][None, :]
    c_ref[0, :, :] = _pack_words(pd, dst)


def _proj_call(dst, Wp, Bp):
    return pl.pallas_call(
        _proj_body,
        grid=(NMP,),
        in_specs=[
            pl.BlockSpec((NROW, D), lambda i: (0, 0)),
            pl.BlockSpec((1, D, D), lambda i: (i, 0, 0)),
            pl.BlockSpec((1, 1, D), lambda i: (i, 0, 0)),
        ],
        out_specs=pl.BlockSpec((1, NROW, D), lambda i: (i, 0, 0)),
        out_shape=jax.ShapeDtypeStruct((NMP, NROW, D), jnp.int32),
    )(dst, Wp, Bp.reshape(NMP, 1, D))


def _gather_body(row0, c_ref, idx_ref, out_ref, idx_v, b0, b1, g0, g1, w0, w1):
    cid = lax.axis_index("c")
    sid = lax.axis_index("s")
    wid = sid * 2 + cid
    pltpu.sync_copy(idx_ref.at[pl.ds(row0 + wid * NCH, NCH)], idx_v)
    bufs = (b0, b1)
    gsems = (g0, g1)
    wsems = (w0, w1)
    PRE = 1

    def gather(c):
        return pltpu.async_copy(c_ref.at[idx_v.at[c]], bufs[c % 2], gsems[c % 2])

    def write(c):
        return pltpu.async_copy(
            bufs[c % 2],
            out_ref.at[pl.ds(wid * (NROW // 2) + (c % 8) * CHUNK, CHUNK),
                       pl.ds((c // 8) * D, D)],
            wsems[c % 2])

    gd, wd, waited = {}, {}, set()
    for c in range(min(PRE, NCH)):
        gd[c] = gather(c)
    for c in range(NCH):
        gd[c].wait()
        wd[c] = write(c)
        n = c + PRE
        if n < NCH:
            prev = n - 2
            if prev >= 0:
                wd[prev].wait()
                waited.add(prev)
            gd[n] = gather(n)
    for c in range(NCH):
        if c not in waited:
            wd[c].wait()


def _gather_call(c_flat, idx2, mp):
    mesh = plsc.VectorSubcoreMesh(core_axis_name="c", subcore_axis_name="s")
    return pl.kernel(
        functools.partial(_gather_body, mp * (K * NROW // CHUNK)),
        out_type=jax.ShapeDtypeStruct((K * NROW // 2, 2 * D), jnp.int32),
        mesh=mesh,
        scratch_types=(
            [pltpu.VMEM((NCH, CHUNK), jnp.int32)]
            + [pltpu.VMEM((CHUNK, D), jnp.int32)] * 2
            + [pltpu.SemaphoreType.DMA] * 4
        ),
        compiler_params=pltpu.CompilerParams(use_tc_tiling_on_sc=False),
    )(c_flat, idx2)


def _main_body(bg_const, ga_ref, nbt_ref, src_ref, V_ref, X_ref, Wq_ref, Bq_ref, Q_ref,
               hs_ref, beta_ref, acc_ref):
    bg = jnp.float32(bg_const)
    j = pl.program_id(0)
    src = src_ref[...]
    su = jnp.dot(src, V_ref[0], preferred_element_type=jnp.float32)
    gw = jnp.concatenate([ga_ref[:, :, 0:D], ga_ref[:, :, D:2 * D]],
                         axis=1)
    g1 = _unpack_lo(gw)
    g2 = _unpack_hi(gw)
    t = jnp.tanh(su[None, :, :] + g1)
    scores = jnp.sum(t * X_ref[0, 0][None, None, :], axis=2)
    nb = nbt_ref[...]
    eq = (nb[:, None, :] == nb[None, :, :]).astype(jnp.float32)
    mult = jnp.sum(eq, axis=1)
    inv_mult = 1.0 / mult
    m = jnp.maximum(jnp.max(scores, axis=0, keepdims=True), bg)
    es = jnp.exp(scores - m)
    n_distinct = jnp.sum(inv_mult, axis=0, keepdims=True)
    denom = jnp.sum(es * inv_mult, axis=0, keepdims=True) \
        + (jnp.float32(NROW) - n_distinct) * jnp.exp(bg - m)
    w = es / denom
    H = src + jnp.sum(w[:, :, None] * g2, axis=0)
    hs_ref[...] = H
    t2 = jnp.tanh(jnp.dot(H, Wq_ref[0], preferred_element_type=jnp.float32)
                  + Bq_ref[0, 0][None, :])
    b = jnp.sum(t2 * Q_ref[0, 0][None, :])

    @pl.when(j == 0)
    def _():
        acc_ref[0] = 0.0

    acc_ref[0] += b

    @pl.when(j == NJB - 1)
    def _():
        beta_ref[0, :] = jnp.full((128,), acc_ref[0], jnp.float32)


def _main_call(ga3, nbT_i, src, V, X, Wq, Bq, Q, mp):
    bg = -1e-9 if mp == 0 else 1.0 / NROW
    return pl.pallas_call(
        functools.partial(_main_body, bg),
        grid=(NJB,),
        in_specs=[
            pl.BlockSpec((K, ROWB // 2, 2 * D), lambda j: (0, j, 0)),
            pl.BlockSpec((K, ROWB), lambda j: (0, j)),
            pl.BlockSpec((ROWB, D), lambda j: (j, 0)),
            pl.BlockSpec((1, D, D), lambda j: (mp, 0, 0)),
            pl.BlockSpec((1, 1, D), lambda j: (mp, 0, 0)),
            pl.BlockSpec((1, D, D), lambda j: (mp, 0, 0)),
            pl.BlockSpec((1, 1, D), lambda j: (mp, 0, 0)),
            pl.BlockSpec((1, 1, D), lambda j: (mp, 0, 0)),
        ],
        out_specs=[
            pl.BlockSpec((ROWB, D), lambda j: (j, 0)),
            pl.BlockSpec((1, 128), lambda j: (0, 0)),
        ],
        out_shape=[
            jax.ShapeDtypeStruct((NROW, D), jnp.float32),
            jax.ShapeDtypeStruct((1, 128), jnp.float32),
        ],
        scratch_shapes=[pltpu.SMEM((1,), jnp.float32)],
    )(ga3, nbT_i, src, V.reshape(NMP, D, D), X.reshape(NMP, 1, D),
      Wq.reshape(NMP, D, D), Bq.reshape(NMP, 1, D), Q.reshape(NMP, 1, D))


def _beta_weights(b0_ref, b1_ref, b2_ref):
    beta = jnp.concatenate(
        [b0_ref[:, 0:1], b1_ref[:, 0:1], b2_ref[:, 0:1]], axis=0) / jnp.float32(NROW)
    m = jnp.max(beta)
    e = jnp.exp(beta - m)
    return e / jnp.sum(e)


def _comb_body(h0_ref, h1_ref, h2_ref, b0_ref, b1_ref, b2_ref, out_ref):
    wv = _beta_weights(b0_ref, b1_ref, b2_ref)
    out_ref[...] = (wv[0, 0] * h0_ref[...] + wv[1, 0] * h1_ref[...]
                    + wv[2, 0] * h2_ref[...])


def _comb_call(hs, betas):
    hspec = pl.BlockSpec((ROWB, D), lambda j: (j, 0))
    bspec = pl.BlockSpec((1, 128), lambda j: (0, 0))
    return pl.pallas_call(
        _comb_body,
        grid=(NJB,),
        in_specs=[hspec, hspec, hspec, bspec, bspec, bspec],
        out_specs=pl.BlockSpec((ROWB, D), lambda j: (j, 0)),
        out_shape=jax.ShapeDtypeStruct((NROW, D), jnp.float32),
    )(*hs, *betas)


def _comb_proj_body(h0_ref, h1_ref, h2_ref, b0_ref, b1_ref, b2_ref,
                    Wp_ref, Bp_ref, c_ref, new_ref):
    i = pl.program_id(0)
    wv = _beta_weights(b0_ref, b1_ref, b2_ref)
    dst = (wv[0, 0] * h0_ref[...] + wv[1, 0] * h1_ref[...]
           + wv[2, 0] * h2_ref[...])
    pd = jnp.dot(dst, Wp_ref[0], preferred_element_type=jnp.float32) + Bp_ref[0, 0][None, :]
    c_ref[0, :, :] = _pack_words(pd, dst)

    @pl.when(i == 0)
    def _():
        new_ref[...] = dst


def _comb_proj_call(hs, betas, Wp, Bp):
    hspec = pl.BlockSpec((NROW, D), lambda i: (0, 0))
    bspec = pl.BlockSpec((1, 128), lambda i: (0, 0))
    return pl.pallas_call(
        _comb_proj_body,
        grid=(NMP,),
        in_specs=[hspec, hspec, hspec, bspec, bspec, bspec,
                  pl.BlockSpec((1, D, D), lambda i: (i, 0, 0)),
                  pl.BlockSpec((1, 1, D), lambda i: (i, 0, 0))],
        out_specs=[
            pl.BlockSpec((1, NROW, D), lambda i: (i, 0, 0)),
            pl.BlockSpec((NROW, D), lambda i: (0, 0)),
        ],
        out_shape=[
            jax.ShapeDtypeStruct((NMP, NROW, D), jnp.int32),
            jax.ShapeDtypeStruct((NROW, D), jnp.float32),
        ],
    )(*hs, *betas, Wp, Bp.reshape(NMP, 1, D))


def _idx2(nbT):
    idx = nbT + (jnp.arange(NMP, dtype=jnp.int32) * NROW)[:, None, None]
    idx = idx.reshape(NMP, K, NJB, 2, CHUNK).transpose(0, 1, 3, 2, 4)
    return idx.reshape(IDXROWS, CHUNK)


def _side_mains(C, idx2, nbT, src, V, X, Wq, Bq, Q):
    c_flat = C.reshape(NMP * NROW, D)
    gas = [_gather_call(c_flat, idx2, i) for i in range(NMP)]
    hs, betas = [], []
    for i in range(NMP):
        ga3 = gas[i].reshape(K, NROW // 2, 2 * D)
        h, b = _main_call(ga3, nbT[i], src, V, X, Wq, Bq, Q, i)
        hs.append(h)
        betas.append(b)
    return hs, betas


def kernel(user, product, user_neighbors, product_neighbors,
           V_u, X_u, Wp_u, Bp_u, Wq_u, Bq_u, Q_u,
           V_p, X_p, Wp_p, Bp_p, Wq_p, Bq_p, Q_p):
    nbT_u = jnp.swapaxes(user_neighbors, 1, 2)
    nbT_p = jnp.swapaxes(product_neighbors, 1, 2)
    C_u = _proj_call(product, Wp_u, Bp_u)
    hs_u, betas_u = _side_mains(C_u, _idx2(nbT_u), nbT_u, user, V_u, X_u, Wq_u, Bq_u, Q_u)
    C_p, user_new = _comb_proj_call(hs_u, betas_u, Wp_p, Bp_p)
    hs_p, betas_p = _side_mains(C_p, _idx2(nbT_p), nbT_p, product, V_p, X_p, Wq_p, Bq_p, Q_p)
    product_new = _comb_call(hs_p, betas_p)
    return (user_new, product_new)


# speedup vs baseline: 1.4206x; 1.4206x over previous
import functools

import jax
import jax.numpy as jnp
from jax import lax
from jax.experimental import pallas as pl
from jax.experimental.pallas import tpu as pltpu
from jax.experimental.pallas import tpu_sc as plsc

NMP = 3
NROW = 2048
D = 64
K = 32
ROWB = 256
NJB = NROW // ROWB
NW = 32
RPW = K * NROW // NW
CHUNK = 128
NCH = RPW // CHUNK
IDXROWS = NMP * K * NROW // CHUNK
HW = D // 2


def _pack_words(pd, dst):
    lo = jax.lax.bitcast_convert_type(pd.astype(jnp.bfloat16), jnp.uint16).astype(jnp.uint32)
    hi = jax.lax.bitcast_convert_type(dst.astype(jnp.bfloat16), jnp.uint16).astype(jnp.uint32)
    return jax.lax.bitcast_convert_type(lo | (hi << 16), jnp.int32)


def _unpack_lo(wds):
    u = jax.lax.bitcast_convert_type(wds, jnp.uint32)
    return jax.lax.bitcast_convert_type((u & 0xFFFF).astype(jnp.uint16),
                                        jnp.bfloat16).astype(jnp.float32)


def _unpack_hi(wds):
    u = jax.lax.bitcast_convert_type(wds, jnp.uint32)
    return jax.lax.bitcast_convert_type((u >> 16).astype(jnp.uint16),
                                        jnp.bfloat16).astype(jnp.float32)


def _proj_body(dst_ref, Wp_ref, Bp_ref, c_ref):
    dst = dst_ref[...]
    pd = jnp.dot(dst, Wp_ref[0], preferred_element_type=jnp.float32) + Bp_ref[0, 0][None, :]
    c_ref[0, :, :] = _pack_words(pd, dst)


def _proj_call(dst, Wp, Bp):
    return pl.pallas_call(
        _proj_body,
        grid=(NMP,),
        in_specs=[
            pl.BlockSpec((NROW, D), lambda i: (0, 0)),
            pl.BlockSpec((1, D, D), lambda i: (i, 0, 0)),
            pl.BlockSpec((1, 1, D), lambda i: (i, 0, 0)),
        ],
        out_specs=pl.BlockSpec((1, NROW, D), lambda i: (i, 0, 0)),
        out_shape=jax.ShapeDtypeStruct((NMP, NROW, D), jnp.int32),
    )(dst, Wp, Bp.reshape(NMP, 1, D))


def _gather_body(row0, c_ref, idx_ref, out_ref, idx_v, b0, b1, g0, g1, w0, w1):
    cid = lax.axis_index("c")
    sid = lax.axis_index("s")
    wid = sid * 2 + cid
    pltpu.sync_copy(idx_ref.at[pl.ds(row0 + wid * NCH, NCH)], idx_v)
    bufs = (b0, b1)
    gsems = (g0, g1)
    wsems = (w0, w1)
    PRE = 1

    def gather(c):
        return pltpu.async_copy(c_ref.at[idx_v.at[c]], bufs[c % 2], gsems[c % 2])

    def write(c):
        return pltpu.async_copy(
            bufs[c % 2],
            out_ref.at[pl.ds((wid % 16) * NROW + (wid // 16) * (NROW // 2)
                             + (c % 8) * CHUNK, CHUNK),
                       pl.ds((c // 8) * D, D)],
            wsems[c % 2])

    gd, wd, waited = {}, {}, set()
    for c in range(min(PRE, NCH)):
        gd[c] = gather(c)
    for c in range(NCH):
        gd[c].wait()
        wd[c] = write(c)
        n = c + PRE
        if n < NCH:
            prev = n - 2
            if prev >= 0:
                wd[prev].wait()
                waited.add(prev)
            gd[n] = gather(n)
    for c in range(NCH):
        if c not in waited:
            wd[c].wait()


def _gather_call(c_flat, idx2, mp):
    mesh = plsc.VectorSubcoreMesh(core_axis_name="c", subcore_axis_name="s")
    return pl.kernel(
        functools.partial(_gather_body, mp * (K * NROW // CHUNK)),
        out_type=jax.ShapeDtypeStruct((K // 2 * NROW, 2 * D), jnp.int32),
        mesh=mesh,
        scratch_types=(
            [pltpu.VMEM((NCH, CHUNK), jnp.int32)]
            + [pltpu.VMEM((CHUNK, D), jnp.int32)] * 2
            + [pltpu.SemaphoreType.DMA] * 4
        ),
        compiler_params=pltpu.CompilerParams(use_tc_tiling_on_sc=False),
    )(c_flat, idx2)


def _main_body(bg_const, ga_ref, nbt_ref, src_ref, V_ref, X_ref, Wq_ref, Bq_ref, Q_ref,
               hs_ref, beta_ref, acc_ref):
    bg = jnp.float32(bg_const)
    j = pl.program_id(0)
    src = src_ref[...]
    su = jnp.dot(src, V_ref[0], preferred_element_type=jnp.float32)
    gw = ga_ref[...]
    g1 = _unpack_lo(gw)
    g2 = _unpack_hi(gw)
    su2 = jnp.concatenate([su, su], axis=1)
    X1 = X_ref[0, 0]
    X2 = jnp.concatenate([X1, X1])
    X2s = jnp.concatenate([X1, -X1])
    t = jnp.tanh(su2[None, :, :] + g1)
    whole = jnp.sum(t * X2[None, None, :], axis=2)
    diff = jnp.sum(t * X2s[None, None, :], axis=2)
    scores = jnp.concatenate([0.5 * (whole + diff),
                              0.5 * (whole - diff)], axis=0)
    nb = nbt_ref[...]
    eq = (nb[:, None, :] == nb[None, :, :]).astype(jnp.float32)
    mult = jnp.sum(eq, axis=1)
    inv_mult = 1.0 / mult
    m = jnp.maximum(jnp.max(scores, axis=0, keepdims=True), bg)
    es = jnp.exp(scores - m)
    n_distinct = jnp.sum(inv_mult, axis=0, keepdims=True)
    denom = jnp.sum(es * inv_mult, axis=0, keepdims=True) \
        + (jnp.float32(NROW) - n_distinct) * jnp.exp(bg - m)
    w = es / denom
    w2 = jnp.concatenate(
        [jnp.broadcast_to(w[0:K // 2, :, None], (K // 2, ROWB, D)),
         jnp.broadcast_to(w[K // 2:K, :, None], (K // 2, ROWB, D))],
        axis=2)
    hd2 = jnp.sum(w2 * g2, axis=0)
    H = src + hd2[:, 0:D] + hd2[:, D:2 * D]
    hs_ref[...] = H
    t2 = jnp.tanh(jnp.dot(H, Wq_ref[0], preferred_element_type=jnp.float32)
                  + Bq_ref[0, 0][None, :])
    b = jnp.sum(t2 * Q_ref[0, 0][None, :])

    @pl.when(j == 0)
    def _():
        acc_ref[0] = 0.0

    acc_ref[0] += b

    @pl.when(j == NJB - 1)
    def _():
        beta_ref[0, :] = jnp.full((128,), acc_ref[0], jnp.float32)


def _main_call(ga3, nbT_i, src, V, X, Wq, Bq, Q, mp):
    bg = -1e-9 if mp == 0 else 1.0 / NROW
    return pl.pallas_call(
        functools.partial(_main_body, bg),
        grid=(NJB,),
        in_specs=[
            pl.BlockSpec((K // 2, ROWB, 2 * D), lambda j: (0, j, 0)),
            pl.BlockSpec((K, ROWB), lambda j: (0, j)),
            pl.BlockSpec((ROWB, D), lambda j: (j, 0)),
            pl.BlockSpec((1, D, D), lambda j: (mp, 0, 0)),
            pl.BlockSpec((1, 1, D), lambda j: (mp, 0, 0)),
            pl.BlockSpec((1, D, D), lambda j: (mp, 0, 0)),
            pl.BlockSpec((1, 1, D), lambda j: (mp, 0, 0)),
            pl.BlockSpec((1, 1, D), lambda j: (mp, 0, 0)),
        ],
        out_specs=[
            pl.BlockSpec((ROWB, D), lambda j: (j, 0)),
            pl.BlockSpec((1, 128), lambda j: (0, 0)),
        ],
        out_shape=[
            jax.ShapeDtypeStruct((NROW, D), jnp.float32),
            jax.ShapeDtypeStruct((1, 128), jnp.float32),
        ],
        scratch_shapes=[pltpu.SMEM((1,), jnp.float32)],
    )(ga3, nbT_i, src, V.reshape(NMP, D, D), X.reshape(NMP, 1, D),
      Wq.reshape(NMP, D, D), Bq.reshape(NMP, 1, D), Q.reshape(NMP, 1, D))


def _beta_weights(b0_ref, b1_ref, b2_ref):
    beta = jnp.concatenate(
        [b0_ref[:, 0:1], b1_ref[:, 0:1], b2_ref[:, 0:1]], axis=0) / jnp.float32(NROW)
    m = jnp.max(beta)
    e = jnp.exp(beta - m)
    return e / jnp.sum(e)


def _comb_body(h0_ref, h1_ref, h2_ref, b0_ref, b1_ref, b2_ref, out_ref):
    wv = _beta_weights(b0_ref, b1_ref, b2_ref)
    out_ref[...] = (wv[0, 0] * h0_ref[...] + wv[1, 0] * h1_ref[...]
                    + wv[2, 0] * h2_ref[...])


def _comb_call(hs, betas):
    hspec = pl.BlockSpec((ROWB, D), lambda j: (j, 0))
    bspec = pl.BlockSpec((1, 128), lambda j: (0, 0))
    return pl.pallas_call(
        _comb_body,
        grid=(NJB,),
        in_specs=[hspec, hspec, hspec, bspec, bspec, bspec],
        out_specs=pl.BlockSpec((ROWB, D), lambda j: (j, 0)),
        out_shape=jax.ShapeDtypeStruct((NROW, D), jnp.float32),
    )(*hs, *betas)


def _comb_proj_body(h0_ref, h1_ref, h2_ref, b0_ref, b1_ref, b2_ref,
                    Wp_ref, Bp_ref, c_ref, new_ref):
    i = pl.program_id(0)
    wv = _beta_weights(b0_ref, b1_ref, b2_ref)
    dst = (wv[0, 0] * h0_ref[...] + wv[1, 0] * h1_ref[...]
           + wv[2, 0] * h2_ref[...])
    pd = jnp.dot(dst, Wp_ref[0], preferred_element_type=jnp.float32) + Bp_ref[0, 0][None, :]
    c_ref[0, :, :] = _pack_words(pd, dst)

    @pl.when(i == 0)
    def _():
        new_ref[...] = dst


def _comb_proj_call(hs, betas, Wp, Bp):
    hspec = pl.BlockSpec((NROW, D), lambda i: (0, 0))
    bspec = pl.BlockSpec((1, 128), lambda i: (0, 0))
    return pl.pallas_call(
        _comb_proj_body,
        grid=(NMP,),
        in_specs=[hspec, hspec, hspec, bspec, bspec, bspec,
                  pl.BlockSpec((1, D, D), lambda i: (i, 0, 0)),
                  pl.BlockSpec((1, 1, D), lambda i: (i, 0, 0))],
        out_specs=[
            pl.BlockSpec((1, NROW, D), lambda i: (i, 0, 0)),
            pl.BlockSpec((NROW, D), lambda i: (0, 0)),
        ],
        out_shape=[
            jax.ShapeDtypeStruct((NMP, NROW, D), jnp.int32),
            jax.ShapeDtypeStruct((NROW, D), jnp.float32),
        ],
    )(*hs, *betas, Wp, Bp.reshape(NMP, 1, D))


def _idx2(nbT):
    idx = nbT + (jnp.arange(NMP, dtype=jnp.int32) * NROW)[:, None, None]
    idx = idx.reshape(NMP, 2, 16, 2, NJB, CHUNK).transpose(0, 3, 2, 1, 4, 5)
    return idx.reshape(IDXROWS, CHUNK)


def _side_mains(C, idx2, nbT, src, V, X, Wq, Bq, Q):
    c_flat = C.reshape(NMP * NROW, D)
    gas = [_gather_call(c_flat, idx2, i) for i in range(NMP)]
    hs, betas = [], []
    for i in range(NMP):
        ga3 = gas[i].reshape(K // 2, NROW, 2 * D)
        h, b = _main_call(ga3, nbT[i], src, V, X, Wq, Bq, Q, i)
        hs.append(h)
        betas.append(b)
    return hs, betas


def kernel(user, product, user_neighbors, product_neighbors,
           V_u, X_u, Wp_u, Bp_u, Wq_u, Bq_u, Q_u,
           V_p, X_p, Wp_p, Bp_p, Wq_p, Bq_p, Q_p):
    nbT_u = jnp.swapaxes(user_neighbors, 1, 2)
    nbT_p = jnp.swapaxes(product_neighbors, 1, 2)
    C_u = _proj_call(product, Wp_u, Bp_u)
    hs_u, betas_u = _side_mains(C_u, _idx2(nbT_u), nbT_u, user, V_u, X_u, Wq_u, Bq_u, Q_u)
    C_p, user_new = _comb_proj_call(hs_u, betas_u, Wp_p, Bp_p)
    hs_p, betas_p = _side_mains(C_p, _idx2(nbT_p), nbT_p, product, V_p, X_p, Wq_p, Bq_p, Q_p)
    product_new = _comb_call(hs_p, betas_p)
    return (user_new, product_new)
